# Initial kernel scaffold; baseline (speedup 1.0000x reference)
import functools
import jax
import jax.numpy as jnp
from jax import lax
from jax.experimental import pallas as pl
from jax.experimental.pallas import tpu as pltpu

_M = 0.2
_LYMBDA = 1.0
_NEG = float("-inf")


def _sig(x):
    return 1.0 / (1.0 + jnp.exp(-x))


def _fused_body(inters_ref, rels_ref, labels_ref, mem_ref, rl_ref, gt_ref,
                mw_ref, out_ref, *, bb, t, c, nr, inv_b):
    x = inters_ref[...]
    memb = mem_ref[...] > 0
    lab = labels_ref[...][:, 0]
    mw = mw_ref[...] > 0

    citer = lax.broadcasted_iota(jnp.int32, (bb, t, c), 2)
    tgt = citer == lab[:, None, None]
    negmask = memb[:, :, None] & mw[:, None, :] & (~tgt)
    m1 = jnp.max(jnp.where(negmask, x, _NEG), axis=2)
    xl = jnp.max(jnp.where(tgt & memb[:, :, None], x, _NEG), axis=2)

    r = rels_ref[...]
    rl = rl_ref[...]
    rowflag = memb & (rl != nr)
    g0 = gt_ref[...][:, 0]
    g1 = gt_ref[...][:, 1]
    titer = lax.broadcasted_iota(jnp.int32, (bb, t), 1)
    rel_t0 = jnp.sum(jnp.where(titer == g0[:, None], rl, 0), axis=1)
    rel_t1 = jnp.sum(jnp.where(titer == g1[:, None], rl, 0), axis=1)

    riter = lax.broadcasted_iota(jnp.int32, (bb, t, nr), 2)
    rneg = (rowflag[:, :, None] & (riter != rel_t0[:, None, None])
            & (riter != rel_t1[:, None, None]))
    m2 = jnp.max(jnp.where(rneg, r, _NEG), axis=2)
    xr = jnp.max(jnp.where(rowflag[:, :, None]
                           & (riter == rel_t0[:, None, None]), r, _NEG),
                 axis=2)

    s_xl = _sig(xl)
    s_xr = _sig(xr)
    mv = (s_xl + s_xr) * memb.astype(jnp.float32)
    maxv = jnp.max(mv, axis=1)
    ismax = mv == maxv[:, None]
    first = jnp.min(jnp.where(ismax, titer, t), axis=1)
    sel = titer == first[:, None]
    pos = jnp.max(jnp.where(sel, s_xl, 0.0), axis=1)
    pos_r = jnp.max(jnp.where(sel, s_xr, 0.0), axis=1)

    term = (_LYMBDA * jnp.maximum(_M - pos[:, None] + _sig(m1), 0.0)
            + jnp.maximum(_M - pos_r[:, None] + _sig(m2), 0.0))
    partial = jnp.sum(term) * inv_b

    @pl.when(pl.program_id(0) == 0)
    def _init():
        out_ref[0, 0] = 0.0

    out_ref[0, 0] += partial


@jax.jit
def kernel(inters, rels, labels, mem_mask, rels_label, gt_tracks,
           multilab_weights):
    b, t, c = inters.shape
    nr = rels.shape[2]
    bb = 8
    grid = (b // bb,)
    body = functools.partial(_fused_body, bb=bb, t=t, c=c, nr=nr,
                             inv_b=1.0 / b)
    out = pl.pallas_call(
        body,
        grid=grid,
        in_specs=[
            pl.BlockSpec((bb, t, c), lambda i: (i, 0, 0)),
            pl.BlockSpec((bb, t, nr), lambda i: (i, 0, 0)),
            pl.BlockSpec((bb, 1), lambda i: (i, 0)),
            pl.BlockSpec((bb, t), lambda i: (i, 0)),
            pl.BlockSpec((bb, t), lambda i: (i, 0)),
            pl.BlockSpec((bb, 2), lambda i: (i, 0)),
            pl.BlockSpec((bb, c), lambda i: (i, 0)),
        ],
        out_specs=pl.BlockSpec((1, 1), lambda i: (0, 0)),
        out_shape=jax.ShapeDtypeStruct((1, 1), jnp.float32),
    )(inters, rels, labels[:, None], mem_mask, rels_label, gt_tracks,
      multilab_weights)
    return out.reshape((1,))


# speedup vs baseline: 1.9442x; 1.9442x over previous
import functools
import jax
import jax.numpy as jnp
from jax import lax
from jax.experimental import pallas as pl
from jax.experimental.pallas import tpu as pltpu

_M = 0.2
_LYMBDA = 1.0
_NEG = float("-inf")


def _sig(x):
    return 1.0 / (1.0 + jnp.exp(-x))


def _fused_body(inters_ref, rels_ref, labels_ref, mem_ref, rl_ref, gt_ref,
                mw_ref, out_ref, *, bb, t, c, nr, inv_b):
    x = inters_ref[...]
    memi = mem_ref[...]
    memb = memi > 0
    mem3 = memi[:, :, None] > 0
    lab = labels_ref[...][:, 0]
    mw3 = mw_ref[...][:, None, :] > 0

    citer = lax.broadcasted_iota(jnp.int32, (bb, t, c), 2)
    tgt = citer == lab[:, None, None]
    negmask = mem3 & mw3 & (~tgt)
    m1 = jnp.max(jnp.where(negmask, x, _NEG), axis=2)
    xl = jnp.max(jnp.where(tgt & mem3, x, _NEG), axis=2)

    r = rels_ref[...]
    rl = rl_ref[...]
    rfi = memi * (rl != nr).astype(jnp.int32)
    rf3 = rfi[:, :, None] > 0
    g0 = gt_ref[...][:, 0]
    g1 = gt_ref[...][:, 1]
    titer = lax.broadcasted_iota(jnp.int32, (bb, t), 1)
    rel_t0 = jnp.sum(jnp.where(titer == g0[:, None], rl, 0), axis=1)
    rel_t1 = jnp.sum(jnp.where(titer == g1[:, None], rl, 0), axis=1)

    riter = lax.broadcasted_iota(jnp.int32, (bb, t, nr), 2)
    rneg = (rf3 & (riter != rel_t0[:, None, None])
            & (riter != rel_t1[:, None, None]))
    m2 = jnp.max(jnp.where(rneg, r, _NEG), axis=2)
    xr = jnp.max(jnp.where(rf3 & (riter == rel_t0[:, None, None]), r, _NEG),
                 axis=2)

    s_xl = _sig(xl)
    s_xr = _sig(xr)
    mv = (s_xl + s_xr) * memb.astype(jnp.float32)
    maxv = jnp.max(mv, axis=1)
    ismax = mv == maxv[:, None]
    first = jnp.min(jnp.where(ismax, titer, t), axis=1)
    sel = titer == first[:, None]
    pos = jnp.max(jnp.where(sel, s_xl, 0.0), axis=1)
    pos_r = jnp.max(jnp.where(sel, s_xr, 0.0), axis=1)

    term = (_LYMBDA * jnp.maximum(_M - pos[:, None] + _sig(m1), 0.0)
            + jnp.maximum(_M - pos_r[:, None] + _sig(m2), 0.0))
    partial = jnp.full((1, 1), jnp.sum(term) * inv_b, jnp.float32)

    @pl.when(pl.program_id(0) == 0)
    def _init():
        out_ref[...] = jnp.zeros((1, 1), jnp.float32)

    out_ref[...] += partial


@jax.jit
def kernel(inters, rels, labels, mem_mask, rels_label, gt_tracks,
           multilab_weights):
    b, t, c = inters.shape
    nr = rels.shape[2]
    bb = 8
    grid = (b // bb,)
    body = functools.partial(_fused_body, bb=bb, t=t, c=c, nr=nr,
                             inv_b=1.0 / b)
    out = pl.pallas_call(
        body,
        grid=grid,
        in_specs=[
            pl.BlockSpec((bb, t, c), lambda i: (i, 0, 0)),
            pl.BlockSpec((bb, t, nr), lambda i: (i, 0, 0)),
            pl.BlockSpec((bb, 1), lambda i: (i, 0)),
            pl.BlockSpec((bb, t), lambda i: (i, 0)),
            pl.BlockSpec((bb, t), lambda i: (i, 0)),
            pl.BlockSpec((bb, 2), lambda i: (i, 0)),
            pl.BlockSpec((bb, c), lambda i: (i, 0)),
        ],
        out_specs=pl.BlockSpec((1, 1), lambda i: (0, 0)),
        out_shape=jax.ShapeDtypeStruct((1, 1), jnp.float32),
    )(inters, rels, labels[:, None], mem_mask, rels_label, gt_tracks,
      multilab_weights)
    return out.reshape((1,))


# speedup vs baseline: 2.7583x; 1.4187x over previous
import functools
import jax
import jax.numpy as jnp
from jax import lax
from jax.experimental import pallas as pl
from jax.experimental.pallas import tpu as pltpu

_M = 0.2
_LYMBDA = 1.0
_NEG = float("-inf")


def _sig(x):
    return 1.0 / (1.0 + jnp.exp(-x))


def _fused_body(inters_ref, rels_ref, labels_ref, mem_ref, rl_ref, gt_ref,
                mw_ref, out_ref, *, bb, t, c, nr, inv_b):
    x = inters_ref[...]
    memi = mem_ref[...]
    memb = memi > 0
    mem3 = memi[:, :, None] > 0
    lab = labels_ref[...][:, 0]
    mw3 = mw_ref[...][:, None, :] > 0

    citer = lax.broadcasted_iota(jnp.int32, (bb, t, c), 2)
    tgt = citer == lab[:, None, None]
    negmask = mem3 & mw3 & (~tgt)
    m1 = jnp.max(jnp.where(negmask, x, _NEG), axis=2)
    xl = jnp.max(jnp.where(tgt & mem3, x, _NEG), axis=2)

    r = rels_ref[...]
    rl = rl_ref[...]
    rfi = memi * (rl != nr).astype(jnp.int32)
    rf3 = rfi[:, :, None] > 0
    g0 = gt_ref[...][:, 0]
    g1 = gt_ref[...][:, 1]
    titer = lax.broadcasted_iota(jnp.int32, (bb, t), 1)
    rel_t0 = jnp.sum(jnp.where(titer == g0[:, None], rl, 0), axis=1)
    rel_t1 = jnp.sum(jnp.where(titer == g1[:, None], rl, 0), axis=1)

    riter = lax.broadcasted_iota(jnp.int32, (bb, t, nr), 2)
    rneg = (rf3 & (riter != rel_t0[:, None, None])
            & (riter != rel_t1[:, None, None]))
    m2 = jnp.max(jnp.where(rneg, r, _NEG), axis=2)
    xr = jnp.max(jnp.where(rf3 & (riter == rel_t0[:, None, None]), r, _NEG),
                 axis=2)

    s_xl = _sig(xl)
    s_xr = _sig(xr)
    mv = (s_xl + s_xr) * memb.astype(jnp.float32)
    maxv = jnp.max(mv, axis=1)
    ismax = mv == maxv[:, None]
    first = jnp.min(jnp.where(ismax, titer, t), axis=1)
    sel = titer == first[:, None]
    pos = jnp.max(jnp.where(sel, s_xl, 0.0), axis=1)
    pos_r = jnp.max(jnp.where(sel, s_xr, 0.0), axis=1)

    term = (_LYMBDA * jnp.maximum(_M - pos[:, None] + _sig(m1), 0.0)
            + jnp.maximum(_M - pos_r[:, None] + _sig(m2), 0.0))
    partial = jnp.full((1, 1), jnp.sum(term) * inv_b, jnp.float32)

    @pl.when(pl.program_id(0) == 0)
    def _init():
        out_ref[...] = jnp.zeros((1, 1), jnp.float32)

    out_ref[...] += partial


@jax.jit
def kernel(inters, rels, labels, mem_mask, rels_label, gt_tracks,
           multilab_weights):
    b, t, c = inters.shape
    nr = rels.shape[2]
    bb = 32
    grid = (b // bb,)
    body = functools.partial(_fused_body, bb=bb, t=t, c=c, nr=nr,
                             inv_b=1.0 / b)
    out = pl.pallas_call(
        body,
        grid=grid,
        in_specs=[
            pl.BlockSpec((bb, t, c), lambda i: (i, 0, 0)),
            pl.BlockSpec((bb, t, nr), lambda i: (i, 0, 0)),
            pl.BlockSpec((bb, 1), lambda i: (i, 0)),
            pl.BlockSpec((bb, t), lambda i: (i, 0)),
            pl.BlockSpec((bb, t), lambda i: (i, 0)),
            pl.BlockSpec((bb, 2), lambda i: (i, 0)),
            pl.BlockSpec((bb, c), lambda i: (i, 0)),
        ],
        out_specs=pl.BlockSpec((1, 1), lambda i: (0, 0)),
        out_shape=jax.ShapeDtypeStruct((1, 1), jnp.float32),
    )(inters, rels, labels[:, None], mem_mask, rels_label, gt_tracks,
      multilab_weights)
    return out.reshape((1,))


# speedup vs baseline: 2.8902x; 1.0478x over previous
import functools
import jax
import jax.numpy as jnp
from jax import lax
from jax.experimental import pallas as pl
from jax.experimental.pallas import tpu as pltpu

_M = 0.2
_LYMBDA = 1.0
_NEG = float("-inf")


def _sig(x):
    return 1.0 / (1.0 + jnp.exp(-x))


def _fused_body(inters_ref, rels_ref, labels_ref, mem_ref, rl_ref, gt_ref,
                mw_ref, out_ref, *, bb, t, c, nr, inv_b):
    x = inters_ref[...]
    memi = mem_ref[...]
    memb = memi > 0
    mem3 = memi[:, :, None] > 0
    lab = labels_ref[...][:, 0]
    mw3 = mw_ref[...][:, None, :] > 0

    citer = lax.broadcasted_iota(jnp.int32, (bb, t, c), 2)
    tgt = citer == lab[:, None, None]
    negmask = mem3 & mw3 & (~tgt)
    m1 = jnp.max(jnp.where(negmask, x, _NEG), axis=2)
    xl = jnp.max(jnp.where(tgt & mem3, x, _NEG), axis=2)

    r = rels_ref[...]
    rl = rl_ref[...]
    rfi = memi * (rl != nr).astype(jnp.int32)
    rf3 = rfi[:, :, None] > 0
    g0 = gt_ref[...][:, 0]
    g1 = gt_ref[...][:, 1]
    titer = lax.broadcasted_iota(jnp.int32, (bb, t), 1)
    rel_t0 = jnp.sum(jnp.where(titer == g0[:, None], rl, 0), axis=1)
    rel_t1 = jnp.sum(jnp.where(titer == g1[:, None], rl, 0), axis=1)

    riter = lax.broadcasted_iota(jnp.int32, (bb, t, nr), 2)
    rneg = (rf3 & (riter != rel_t0[:, None, None])
            & (riter != rel_t1[:, None, None]))
    m2 = jnp.max(jnp.where(rneg, r, _NEG), axis=2)
    xr = jnp.max(jnp.where(rf3 & (riter == rel_t0[:, None, None]), r, _NEG),
                 axis=2)

    s_xl = _sig(xl)
    s_xr = _sig(xr)
    mv = (s_xl + s_xr) * memb.astype(jnp.float32)
    maxv = jnp.max(mv, axis=1)
    ismax = mv == maxv[:, None]
    first = jnp.min(jnp.where(ismax, titer, t), axis=1)
    sel = titer == first[:, None]
    pos = jnp.max(jnp.where(sel, s_xl, 0.0), axis=1)
    pos_r = jnp.max(jnp.where(sel, s_xr, 0.0), axis=1)

    term = (_LYMBDA * jnp.maximum(_M - pos[:, None] + _sig(m1), 0.0)
            + jnp.maximum(_M - pos_r[:, None] + _sig(m2), 0.0))
    partial = jnp.full((1, 1), jnp.sum(term) * inv_b, jnp.float32)

    @pl.when(pl.program_id(0) == 0)
    def _init():
        out_ref[...] = jnp.zeros((1, 1), jnp.float32)

    out_ref[...] += partial


@jax.jit
def kernel(inters, rels, labels, mem_mask, rels_label, gt_tracks,
           multilab_weights):
    b, t, c = inters.shape
    nr = rels.shape[2]
    bb = 64
    grid = (b // bb,)
    body = functools.partial(_fused_body, bb=bb, t=t, c=c, nr=nr,
                             inv_b=1.0 / b)
    out = pl.pallas_call(
        body,
        grid=grid,
        in_specs=[
            pl.BlockSpec((bb, t, c), lambda i: (i, 0, 0)),
            pl.BlockSpec((bb, t, nr), lambda i: (i, 0, 0)),
            pl.BlockSpec((bb, 1), lambda i: (i, 0)),
            pl.BlockSpec((bb, t), lambda i: (i, 0)),
            pl.BlockSpec((bb, t), lambda i: (i, 0)),
            pl.BlockSpec((bb, 2), lambda i: (i, 0)),
            pl.BlockSpec((bb, c), lambda i: (i, 0)),
        ],
        out_specs=pl.BlockSpec((1, 1), lambda i: (0, 0)),
        out_shape=jax.ShapeDtypeStruct((1, 1), jnp.float32),
    )(inters, rels, labels[:, None], mem_mask, rels_label, gt_tracks,
      multilab_weights)
    return out.reshape((1,))


# speedup vs baseline: 3.8691x; 1.3387x over previous
import functools
import jax
import jax.numpy as jnp
from jax import lax
from jax.experimental import pallas as pl
from jax.experimental.pallas import tpu as pltpu

_M = 0.2
_LYMBDA = 1.0
_NEG = float("-inf")


def _sig(x):
    return 1.0 / (1.0 + jnp.exp(-x))


def _fused_body(inters_ref, rels_ref, labels_ref, mem_ref, rl_ref, gt_ref,
                mw_ref, out_ref, *, bb, t, c, nr, inv_b):
    x = inters_ref[...]
    memi = mem_ref[...]
    memb = memi > 0
    lab = labels_ref[...][:, 0]

    citer2 = lax.broadcasted_iota(jnp.int32, (bb, c), 1)
    is_lab = citer2 == lab[:, None]
    cbias1 = jnp.where((mw_ref[...] > 0) & (~is_lab), 0.0, _NEG)
    cbias_l = jnp.where(is_lab, 0.0, _NEG)
    tbias = jnp.where(memb, 0.0, _NEG)

    m1 = jnp.max(x + cbias1[:, None, :], axis=2) + tbias
    xl = jnp.max(x + cbias_l[:, None, :], axis=2) + tbias

    r = rels_ref[...]
    rl = rl_ref[...]
    g0 = gt_ref[...][:, 0]
    g1 = gt_ref[...][:, 1]
    titer = lax.broadcasted_iota(jnp.int32, (bb, t), 1)
    rel_t0 = jnp.sum(jnp.where(titer == g0[:, None], rl, 0), axis=1)
    rel_t1 = jnp.sum(jnp.where(titer == g1[:, None], rl, 0), axis=1)

    riter2 = lax.broadcasted_iota(jnp.int32, (bb, nr), 1)
    rbias = jnp.where((riter2 != rel_t0[:, None])
                      & (riter2 != rel_t1[:, None]), 0.0, _NEG)
    xrbias = jnp.where(riter2 == rel_t0[:, None], 0.0, _NEG)
    rtbias = jnp.where(memb & (rl != nr), 0.0, _NEG)

    m2 = jnp.max(r + rbias[:, None, :], axis=2) + rtbias
    xr = jnp.max(r + xrbias[:, None, :], axis=2) + rtbias

    s_xl = _sig(xl)
    s_xr = _sig(xr)
    mv = (s_xl + s_xr) * memb.astype(jnp.float32)
    maxv = jnp.max(mv, axis=1)
    ismax = mv == maxv[:, None]
    first = jnp.min(jnp.where(ismax, titer, t), axis=1)
    sel = titer == first[:, None]
    pos = jnp.max(jnp.where(sel, s_xl, 0.0), axis=1)
    pos_r = jnp.max(jnp.where(sel, s_xr, 0.0), axis=1)

    term = (_LYMBDA * jnp.maximum(_M - pos[:, None] + _sig(m1), 0.0)
            + jnp.maximum(_M - pos_r[:, None] + _sig(m2), 0.0))
    partial = jnp.full((1, 1), jnp.sum(term) * inv_b, jnp.float32)

    @pl.when(pl.program_id(0) == 0)
    def _init():
        out_ref[...] = jnp.zeros((1, 1), jnp.float32)

    out_ref[...] += partial


@jax.jit
def kernel(inters, rels, labels, mem_mask, rels_label, gt_tracks,
           multilab_weights):
    b, t, c = inters.shape
    nr = rels.shape[2]
    bb = 64
    grid = (b // bb,)
    body = functools.partial(_fused_body, bb=bb, t=t, c=c, nr=nr,
                             inv_b=1.0 / b)
    out = pl.pallas_call(
        body,
        grid=grid,
        in_specs=[
            pl.BlockSpec((bb, t, c), lambda i: (i, 0, 0)),
            pl.BlockSpec((bb, t, nr), lambda i: (i, 0, 0)),
            pl.BlockSpec((bb, 1), lambda i: (i, 0)),
            pl.BlockSpec((bb, t), lambda i: (i, 0)),
            pl.BlockSpec((bb, t), lambda i: (i, 0)),
            pl.BlockSpec((bb, 2), lambda i: (i, 0)),
            pl.BlockSpec((bb, c), lambda i: (i, 0)),
        ],
        out_specs=pl.BlockSpec((1, 1), lambda i: (0, 0)),
        out_shape=jax.ShapeDtypeStruct((1, 1), jnp.float32),
    )(inters, rels, labels[:, None], mem_mask, rels_label, gt_tracks,
      multilab_weights)
    return out.reshape((1,))
